# baseline (speedup 1.0000x reference)
import jax
import jax.numpy as jnp
from jax import lax
from jax.experimental import pallas as pl
from jax.experimental.pallas import tpu as pltpu
from jax.experimental.pallas import tpu_sc as plsc

N = 106496
N_QUANTILES = 1000
N_TOKENS = 4
D = 128

_info = plsc.get_sparse_core_info()
_NC, _NS, _L = _info.num_cores, _info.num_subcores, _info.num_lanes
_NW = _NC * _NS
_BPW = N // _NW
_CHUNK = 128
_NCHUNK = _BPW // _CHUNK
_VECS_PER_CHUNK = _CHUNK // _L


def _body(vals_hbm, inds_hbm, table_hbm, out_hbm,
          vals_v, inds_v, idx_v, rows_v, gsem):
    wid = lax.axis_index("s") * _NC + lax.axis_index("c")
    base = wid * _BPW

    pltpu.sync_copy(vals_hbm.at[pl.ds(base, _BPW)], vals_v)
    pltpu.sync_copy(inds_hbm.at[pl.ds(base, _BPW)], inds_v)

    def compute_chunk(j, _):
        for k in range(_VECS_PER_CHUNK):
            off = j * _CHUNK + k * _L
            v = vals_v[pl.ds(off, _L)]
            ind = inds_v[pl.ds(off, _L)]
            v = jnp.minimum(jnp.maximum(v, 0.0), 1.0)
            q = (v * jnp.float32(N_QUANTILES)).astype(jnp.int32) + N_TOKENS
            q = jnp.minimum(q, N_QUANTILES + N_TOKENS - 1)
            idx_v[j, pl.ds(k * _L, _L)] = jnp.where(ind == 0, q, ind)
        return 0

    lax.fori_loop(0, _NCHUNK, compute_chunk, 0)

    def gather_chunk(j, _):
        pltpu.async_copy(table_hbm.at[idx_v.at[j]], rows_v, gsem).wait()
        pltpu.sync_copy(rows_v, out_hbm.at[pl.ds(base + j * _CHUNK, _CHUNK)])
        return 0

    lax.fori_loop(0, _NCHUNK, gather_chunk, 0)


def kernel(values, indicators, table):
    mesh = plsc.VectorSubcoreMesh(core_axis_name="c", subcore_axis_name="s")
    run = pl.kernel(
        _body,
        mesh=mesh,
        out_type=jax.ShapeDtypeStruct((N, D), jnp.float32),
        scratch_types=[
            pltpu.VMEM((_BPW,), jnp.float32),
            pltpu.VMEM((_BPW,), jnp.int32),
            pltpu.VMEM((_NCHUNK, _CHUNK), jnp.int32),
            pltpu.VMEM((_CHUNK, D), jnp.float32),
            pltpu.SemaphoreType.DMA,
        ],
    )
    return run(values, indicators, table)


# speedup vs baseline: 1.0025x; 1.0025x over previous
import jax
import jax.numpy as jnp
from jax import lax
from jax.experimental import pallas as pl
from jax.experimental.pallas import tpu as pltpu
from jax.experimental.pallas import tpu_sc as plsc

N = 106496
N_QUANTILES = 1000
N_TOKENS = 4
D = 128

_info = plsc.get_sparse_core_info()
_NC, _NS, _L = _info.num_cores, _info.num_subcores, _info.num_lanes
_NW = _NC * _NS
_BPW = N // _NW
_CHUNK = 128
_NCHUNK = _BPW // _CHUNK
_NBUF = 4
_LA = 3


def _body(vals_hbm, inds_hbm, table_hbm, out_hbm,
          vals_v, inds_v, idx_v, rows_v, gsems, wsems):
    wid = lax.axis_index("s") * _NC + lax.axis_index("c")
    base = wid * _BPW

    pltpu.sync_copy(vals_hbm.at[pl.ds(base, _BPW)], vals_v)
    pltpu.sync_copy(inds_hbm.at[pl.ds(base, _BPW)], inds_v)

    def compute_vec(i, _):
        off = i * _L
        v = vals_v[pl.ds(off, _L)]
        ind = inds_v[pl.ds(off, _L)]
        v = jnp.minimum(jnp.maximum(v, 0.0), 1.0)
        q = (v * jnp.float32(N_QUANTILES)).astype(jnp.int32) + N_TOKENS
        q = jnp.minimum(q, N_QUANTILES + N_TOKENS - 1)
        idx_v[pl.ds(off, _L)] = jnp.where(ind == 0, q, ind)
        return 0

    lax.fori_loop(0, _BPW // _L, compute_vec, 0)

    def start_gather(j):
        b = j % _NBUF
        return pltpu.async_copy(
            table_hbm.at[idx_v.at[pl.ds(j * _CHUNK, _CHUNK)]],
            rows_v.at[b], gsems.at[b])

    def start_write(j):
        b = j % _NBUF
        return pltpu.async_copy(
            rows_v.at[b], out_hbm.at[pl.ds(base + j * _CHUNK, _CHUNK)],
            wsems.at[b])

    writes = [None] * _NCHUNK
    gathers = [None] * _NCHUNK
    for t in range(_NCHUNK + _LA):
        if t < _NCHUNK:
            if t >= _NBUF:
                writes[t - _NBUF].wait()
            gathers[t] = start_gather(t)
        if t >= _LA:
            j = t - _LA
            gathers[j].wait()
            writes[j] = start_write(j)
    for j in range(_NCHUNK - _NBUF, _NCHUNK):
        writes[j].wait()


def kernel(values, indicators, table):
    mesh = plsc.VectorSubcoreMesh(core_axis_name="c", subcore_axis_name="s")
    run = pl.kernel(
        _body,
        mesh=mesh,
        out_type=jax.ShapeDtypeStruct((N, D), jnp.float32),
        scratch_types=[
            pltpu.VMEM((_BPW,), jnp.float32),
            pltpu.VMEM((_BPW,), jnp.int32),
            pltpu.VMEM((_BPW,), jnp.int32),
            pltpu.VMEM((_NBUF, _CHUNK, D), jnp.float32),
            pltpu.SemaphoreType.DMA((_NBUF,)),
            pltpu.SemaphoreType.DMA((_NBUF,)),
        ],
    )
    return run(values, indicators, table)
